# Initial kernel scaffold; baseline (speedup 1.0000x reference)
import jax
import jax.numpy as jnp
from jax.experimental import pallas as pl


def kernel(pred, target):
    raise NotImplementedError("write your pallas kernel here")


# speedup vs baseline: 7921.7900x; 7921.7900x over previous
import jax
import jax.numpy as jnp
from jax.experimental import pallas as pl
from jax.experimental.pallas import tpu as pltpu

_N = 4096
_B = 512
_ALPHA = 4.0


def _loss_kernel(pc_ref, tc_ref, pr_ref, tr_ref, out_ref, acc_ref):
    i = pl.program_id(0)

    pi = pc_ref[...]
    ti = tc_ref[...]
    pj = pr_ref[...]
    tj = tr_ref[...]

    dt = ti - tj
    s = jnp.sign(dt)
    m = s * s
    dp = pi - pj
    c = jnp.maximum(m - dp * s, 0.0)

    s_part = jnp.sum(c)
    c_part = jnp.sum(m)
    e = pi - ti
    mse_part = jnp.sum(e * e)

    @pl.when(i == 0)
    def _init():
        acc_ref[0] = 0.0
        acc_ref[1] = 0.0
        acc_ref[2] = 0.0

    acc_ref[0] += s_part
    acc_ref[1] += c_part
    acc_ref[2] += mse_part

    @pl.when(i == pl.num_programs(0) - 1)
    def _finish():
        total = acc_ref[2] / _N + _ALPHA * (acc_ref[0] / jnp.maximum(acc_ref[1], 1.0))
        out_ref[0, 0] = total


@jax.jit
def kernel(pred, target):
    pc = pred.reshape(_N, 1)
    tc = target.reshape(_N, 1)
    pr = pred.reshape(1, _N)
    tr = target.reshape(1, _N)
    out = pl.pallas_call(
        _loss_kernel,
        grid=(_N // _B,),
        in_specs=[
            pl.BlockSpec((_B, 1), lambda i: (i, 0)),
            pl.BlockSpec((_B, 1), lambda i: (i, 0)),
            pl.BlockSpec((1, _N), lambda i: (0, 0)),
            pl.BlockSpec((1, _N), lambda i: (0, 0)),
        ],
        out_specs=pl.BlockSpec(memory_space=pltpu.SMEM),
        out_shape=jax.ShapeDtypeStruct((1, 1), jnp.float32),
        scratch_shapes=[pltpu.SMEM((4,), jnp.float32)],
    )(pc, tc, pr, tr)
    return out[0, 0]
